# Initial kernel scaffold; baseline (speedup 1.0000x reference)
import jax
import jax.numpy as jnp
from jax.experimental import pallas as pl


def kernel(user_indices, item_indices, user_emb_mf, item_emb_mf, user_emb_mlp, item_emb_mlp, W1, b1, W2, b2, W3, b3, Wo, bo):
    raise NotImplementedError("write your pallas kernel here")


# speedup vs baseline: 7.5450x; 7.5450x over previous
import functools

import jax
import jax.numpy as jnp
from jax import lax
from jax.experimental import pallas as pl
from jax.experimental.pallas import tpu as pltpu
from jax.experimental.pallas import tpu_sc as plsc

_NC = 2
_NS = 16
_NW = _NC * _NS

_BATCH = 16384
_D = 128
_ROWS_PER_W = _BATCH // _NW
_CHUNK = 128
_CHUNKS_PER_W = _ROWS_PER_W // _CHUNK


def _sc_gather_body(uidx_hbm, iidx_hbm, t_umf, t_imf, t_umlp, t_imlp,
                    o_umf, o_imf, o_umlp, o_imlp,
                    uidx_v, iidx_v, rows_a, rows_b, sem_a, sem_b):
  wid = lax.axis_index("s") * _NC + lax.axis_index("c")
  idx_row_base = wid * _CHUNKS_PER_W
  out_base = wid * _ROWS_PER_W

  pltpu.sync_copy(uidx_hbm.at[pl.ds(idx_row_base, _CHUNKS_PER_W)], uidx_v)
  pltpu.sync_copy(iidx_hbm.at[pl.ds(idx_row_base, _CHUNKS_PER_W)], iidx_v)

  rounds = []
  for tab, idx_v, out in ((t_umf, uidx_v, o_umf), (t_imf, iidx_v, o_imf),
                          (t_umlp, uidx_v, o_umlp), (t_imlp, iidx_v, o_imlp)):
    for j in range(_CHUNKS_PER_W):
      rounds.append((tab, idx_v, j, out))

  bufs = (rows_a, rows_b)
  sems = (sem_a, sem_b)

  def start(r):
    tab, idx_v, j, _ = rounds[r]
    pltpu.async_copy(tab.at[idx_v.at[j]], bufs[r % 2], sems[r % 2])

  start(0)
  for r in range(len(rounds)):
    if r + 1 < len(rounds):
      start(r + 1)
    _, _, j, out = rounds[r]
    pltpu.make_async_copy(
        rounds[r][0].at[rounds[r][1].at[j]], bufs[r % 2], sems[r % 2]).wait()
    pltpu.sync_copy(bufs[r % 2], out.at[pl.ds(out_base + j * _CHUNK, _CHUNK)])


def _sc_gather(uidx2d, iidx2d, t_umf, t_imf, t_umlp, t_imlp):
  mesh = plsc.VectorSubcoreMesh(core_axis_name="c", subcore_axis_name="s",
                                num_cores=_NC, num_subcores=_NS)
  out = jax.ShapeDtypeStruct((_BATCH, _D), jnp.float32)
  k = pl.kernel(
      _sc_gather_body,
      out_type=(out, out, out, out),
      mesh=mesh,
      scratch_types=[
          pltpu.VMEM((_CHUNKS_PER_W, _CHUNK), jnp.int32),
          pltpu.VMEM((_CHUNKS_PER_W, _CHUNK), jnp.int32),
          pltpu.VMEM((_CHUNK, _D), jnp.float32),
          pltpu.VMEM((_CHUNK, _D), jnp.float32),
          pltpu.SemaphoreType.DMA,
          pltpu.SemaphoreType.DMA,
      ],
  )
  return k(uidx2d, iidx2d, t_umf, t_imf, t_umlp, t_imlp)


_BLK = 1024


def _tc_mlp_body(umf, imf, umlp, imlp, w1a, w1b, b1, w2, b2, w3, b3,
                 womf, womlp, bo, out):
  h = jnp.dot(umlp[...], w1a[...], preferred_element_type=jnp.float32)
  h += jnp.dot(imlp[...], w1b[...], preferred_element_type=jnp.float32)
  h = jnp.maximum(h + b1[...], 0.0)
  h = jnp.maximum(
      jnp.dot(h, w2[...], preferred_element_type=jnp.float32) + b2[...], 0.0)
  h = jnp.maximum(
      jnp.dot(h, w3[...], preferred_element_type=jnp.float32) + b3[...], 0.0)
  logit = jnp.dot(h, womlp[...], preferred_element_type=jnp.float32)
  mf = umf[...] * imf[...]
  logit += jnp.sum(mf * womf[...], axis=1, keepdims=True)
  logit += bo[...]
  out[...] = (1.0 / (1.0 + jnp.exp(-logit)))[:, 0]


def _tc_mlp(umf, imf, umlp, imlp, w1a, w1b, b1, w2, b2, w3, b3, womf,
            womlp, bo):
  n_blk = _BATCH // _BLK
  batch_spec = pl.BlockSpec((_BLK, _D), lambda i: (i, 0))
  full = lambda shape: pl.BlockSpec(shape, lambda i: tuple(0 for _ in shape))
  return pl.pallas_call(
      _tc_mlp_body,
      grid=(n_blk,),
      in_specs=[
          batch_spec, batch_spec, batch_spec, batch_spec,
          full((_D, _D)), full((_D, _D)), full((1, _D)),
          full((_D, 64)), full((1, 64)),
          full((64, 32)), full((1, 32)),
          full((1, _D)), full((32, 1)), full((1, 1)),
      ],
      out_specs=pl.BlockSpec((_BLK,), lambda i: (i,)),
      out_shape=jax.ShapeDtypeStruct((_BATCH,), jnp.float32),
  )(umf, imf, umlp, imlp, w1a, w1b, b1, w2, b2, w3, b3, womf, womlp, bo)


@jax.jit
def kernel(user_indices, item_indices, user_emb_mf, item_emb_mf,
           user_emb_mlp, item_emb_mlp, W1, b1, W2, b2, W3, b3, Wo, bo):
  uidx2d = user_indices.reshape(_BATCH // _CHUNK, _CHUNK)
  iidx2d = item_indices.reshape(_BATCH // _CHUNK, _CHUNK)
  umf, imf, umlp, imlp = _sc_gather(uidx2d, iidx2d, user_emb_mf, item_emb_mf,
                                    user_emb_mlp, item_emb_mlp)
  w1a = W1[:, :_D].T
  w1b = W1[:, _D:].T
  w2 = W2.T
  w3 = W3.T
  womf = Wo[:, :_D]
  womlp = Wo[0, _D:].reshape(32, 1)
  return _tc_mlp(umf, imf, umlp, imlp, w1a, w1b, b1.reshape(1, _D),
                 w2, b2.reshape(1, 64), w3, b3.reshape(1, 32),
                 womf, womlp, bo.reshape(1, 1))
